# baseline (speedup 1.0000x reference)
import functools
import math

import jax
import jax.numpy as jnp
from jax import lax
from jax.experimental import pallas as pl
from jax.experimental.pallas import tpu as pltpu
from jax.experimental.pallas import tpu_sc as plsc


def _make_sc_gather(E, n_nodes, n_edge_types, n_types):
    info = plsc.get_sparse_core_info()
    NC, NS = info.num_cores, info.num_subcores
    NW = NC * NS
    C = E // NW
    assert E % NW == 0 and C % 8 == 0 and C >= 16, E
    n_groups = (C + 15) // 16
    mesh = plsc.VectorSubcoreMesh(core_axis_name="c", subcore_axis_name="s")

    @functools.partial(
        pl.kernel,
        mesh=mesh,
        compiler_params=pltpu.CompilerParams(needs_layout_passes=False),
        out_type=[
            jax.ShapeDtypeStruct((E,), jnp.float32),
            jax.ShapeDtypeStruct((E,), jnp.float32),
            jax.ShapeDtypeStruct((E,), jnp.float32),
        ],
        scratch_types=[
            pltpu.VMEM((n_nodes,), jnp.float32),
            pltpu.VMEM((n_nodes,), jnp.float32),
            pltpu.VMEM((n_nodes,), jnp.float32),
            pltpu.VMEM((n_nodes,), jnp.int32),
            pltpu.VMEM((n_edge_types,), jnp.float32),
            pltpu.VMEM((n_edge_types,), jnp.float32),
            pltpu.VMEM((C,), jnp.int32),
            pltpu.VMEM((C,), jnp.int32),
            pltpu.VMEM((C,), jnp.float32),
            pltpu.VMEM((C,), jnp.float32),
            pltpu.VMEM((C,), jnp.float32),
        ],
    )
    def sc_gather(px_h, py_h, pz_h, atom_h, mulw_h, biasw_h, ei_h, ej_h,
                  d2_h, mul_h, bias_h,
                  px_v, py_v, pz_v, atom_v, mulw_v, biasw_v,
                  ei_v, ej_v, d2_v, mul_v, bias_v):
        wid = lax.axis_index("s") * NC + lax.axis_index("c")
        base = wid * C
        pltpu.sync_copy(px_h, px_v)
        pltpu.sync_copy(py_h, py_v)
        pltpu.sync_copy(pz_h, pz_v)
        pltpu.sync_copy(atom_h, atom_v)
        pltpu.sync_copy(mulw_h, mulw_v)
        pltpu.sync_copy(biasw_h, biasw_v)
        pltpu.sync_copy(ei_h.at[pl.ds(base, C)], ei_v)
        pltpu.sync_copy(ej_h.at[pl.ds(base, C)], ej_v)

        def body(i, carry):
            off = jnp.minimum(i * 16, C - 16)
            ei = ei_v[pl.ds(off, 16)]
            ej = ej_v[pl.ds(off, 16)]
            xi = plsc.load_gather(px_v, [ei])
            yi = plsc.load_gather(py_v, [ei])
            zi = plsc.load_gather(pz_v, [ei])
            xj = plsc.load_gather(px_v, [ej])
            yj = plsc.load_gather(py_v, [ej])
            zj = plsc.load_gather(pz_v, [ej])
            dx = xi - xj
            dy = yi - yj
            dz = zi - zj
            d2 = dx * dx + dy * dy + dz * dz
            ai = plsc.load_gather(atom_v, [ei])
            aj = plsc.load_gather(atom_v, [ej])
            t = ai * n_types + aj
            mul = plsc.load_gather(mulw_v, [t])
            bias = plsc.load_gather(biasw_v, [t])
            d2_v[pl.ds(off, 16)] = d2
            mul_v[pl.ds(off, 16)] = mul
            bias_v[pl.ds(off, 16)] = bias
            return carry

        lax.fori_loop(0, n_groups, body, 0)
        pltpu.sync_copy(d2_v, d2_h.at[pl.ds(base, C)])
        pltpu.sync_copy(mul_v, mul_h.at[pl.ds(base, C)])
        pltpu.sync_copy(bias_v, bias_h.at[pl.ds(base, C)])

    return sc_gather


def _tc_rbf_chunk(d2, mul, bias, means, stds, prev, chunk_idx, n_chunks,
                  E, block_e):
    ch = d2.shape[0]
    G = means.shape[1]
    inv_a = 1.0 / math.sqrt(2.0 * math.pi)
    log2e = math.log2(math.e)
    rows = block_e // G
    nblk = ch // block_e
    nblk_total = E // block_e
    blk0 = chunk_idx * nblk

    def body(d2_ref, mul_ref, bias_ref, means_ref, stds_ref, *rest):
        out_ref, len_ref = rest[-2], rest[-1]
        length_t = jnp.sqrt(d2_ref[0])
        x_t = mul_ref[0] * length_t + bias_ref[0]
        xT = x_t.T
        std = jnp.abs(stds_ref[...]) + 1e-5
        inv = 1.0 / std
        lc = jnp.log2(inv * inv_a)
        neg_half_log2e = -0.5 * log2e
        for r in range(rows):
            col = jax.lax.slice(xT, (0, r), (G, r + 1))
            z = (col - means_ref[...]) * inv
            out_ref[pl.ds(r * G, G), :] = jnp.exp2((z * z) * neg_half_log2e + lc)
        len_ref[0] = length_t

    in_specs = [
        pl.BlockSpec((1, rows, G), lambda i: (i, 0, 0)),
        pl.BlockSpec((1, rows, G), lambda i: (i, 0, 0)),
        pl.BlockSpec((1, rows, G), lambda i: (i, 0, 0)),
        pl.BlockSpec((1, G), lambda i: (0, 0)),
        pl.BlockSpec((1, G), lambda i: (0, 0)),
    ]
    args = [d2.reshape(nblk, rows, G), mul.reshape(nblk, rows, G),
            bias.reshape(nblk, rows, G), means, stds]
    aliases = {}
    if prev is not None:
        in_specs += [pl.BlockSpec(memory_space=pl.ANY),
                     pl.BlockSpec(memory_space=pl.ANY)]
        args += [prev[0], prev[1]]
        aliases = {5: 0, 6: 1}

    return pl.pallas_call(
        body,
        grid=(nblk,),
        in_specs=in_specs,
        out_specs=[
            pl.BlockSpec((block_e, G), lambda i: (i + blk0, 0)),
            pl.BlockSpec((1, rows, G), lambda i: (i + blk0, 0, 0)),
        ],
        out_shape=[
            jax.ShapeDtypeStruct((E, G), jnp.float32),
            jax.ShapeDtypeStruct((nblk_total, rows, G), jnp.float32),
        ],
        input_output_aliases=aliases,
    )(*args)


def kernel(pos, edge_index, atom_ind, means, stds, mul_w, bias_w):
    E = edge_index.shape[1]
    n_nodes = pos.shape[0]
    n_edge_types = mul_w.shape[0]
    n_types = int(round(math.sqrt(n_edge_types)))
    n_chunks = 2
    ch = E // n_chunks
    block_e = 16000

    px, py, pz = pos[:, 0], pos[:, 1], pos[:, 2]
    mw, bw = mul_w.reshape(-1), bias_w.reshape(-1)
    sc = _make_sc_gather(ch, n_nodes, n_edge_types, n_types)

    prev = None
    for c in range(n_chunks):
        lo = c * ch
        d2, mul, bias = sc(px, py, pz, atom_ind, mw, bw,
                           jax.lax.slice(edge_index[0], (lo,), (lo + ch,)),
                           jax.lax.slice(edge_index[1], (lo,), (lo + ch,)))
        prev = _tc_rbf_chunk(d2, mul, bias, means, stds, prev, c, n_chunks,
                             E, block_e)
    out, length = prev
    return out.astype(means.dtype), length.reshape(E, 1)


# speedup vs baseline: 1.1223x; 1.1223x over previous
import functools
import math

import jax
import jax.numpy as jnp
from jax import lax
from jax.experimental import pallas as pl
from jax.experimental.pallas import tpu as pltpu
from jax.experimental.pallas import tpu_sc as plsc


def _make_sc_gather(E, n_nodes, n_edge_types, n_types):
    info = plsc.get_sparse_core_info()
    NC, NS = info.num_cores, info.num_subcores
    NW = NC * NS
    C = E // NW
    assert E % NW == 0 and C % 8 == 0 and C >= 16, E
    n_groups = (C + 15) // 16
    mesh = plsc.VectorSubcoreMesh(core_axis_name="c", subcore_axis_name="s")

    @functools.partial(
        pl.kernel,
        mesh=mesh,
        compiler_params=pltpu.CompilerParams(needs_layout_passes=False),
        out_type=[
            jax.ShapeDtypeStruct((E,), jnp.float32),
            jax.ShapeDtypeStruct((E,), jnp.float32),
            jax.ShapeDtypeStruct((E,), jnp.float32),
        ],
        scratch_types=[
            pltpu.VMEM((n_nodes,), jnp.float32),
            pltpu.VMEM((n_nodes,), jnp.float32),
            pltpu.VMEM((n_nodes,), jnp.float32),
            pltpu.VMEM((n_nodes,), jnp.int32),
            pltpu.VMEM((n_edge_types,), jnp.float32),
            pltpu.VMEM((n_edge_types,), jnp.float32),
            pltpu.VMEM((C,), jnp.int32),
            pltpu.VMEM((C,), jnp.int32),
            pltpu.VMEM((C,), jnp.float32),
            pltpu.VMEM((C,), jnp.float32),
            pltpu.VMEM((C,), jnp.float32),
            pltpu.SemaphoreType.DMA,
        ],
    )
    def sc_gather(px_h, py_h, pz_h, atom_h, mulw_h, biasw_h, ei_h, ej_h,
                  d2_h, mul_h, bias_h,
                  px_v, py_v, pz_v, atom_v, mulw_v, biasw_v,
                  ei_v, ej_v, d2_v, mul_v, bias_v, sem):
        wid = lax.axis_index("s") * NC + lax.axis_index("c")
        base = wid * C
        copies = [
            pltpu.async_copy(px_h, px_v, sem),
            pltpu.async_copy(py_h, py_v, sem),
            pltpu.async_copy(pz_h, pz_v, sem),
            pltpu.async_copy(atom_h, atom_v, sem),
            pltpu.async_copy(mulw_h, mulw_v, sem),
            pltpu.async_copy(biasw_h, biasw_v, sem),
            pltpu.async_copy(ei_h.at[pl.ds(base, C)], ei_v, sem),
            pltpu.async_copy(ej_h.at[pl.ds(base, C)], ej_v, sem),
        ]
        for cp in copies:
            cp.wait()

        def body(i, carry):
            off = jnp.minimum(i * 16, C - 16)
            ei = ei_v[pl.ds(off, 16)]
            ej = ej_v[pl.ds(off, 16)]
            xi = plsc.load_gather(px_v, [ei])
            yi = plsc.load_gather(py_v, [ei])
            zi = plsc.load_gather(pz_v, [ei])
            xj = plsc.load_gather(px_v, [ej])
            yj = plsc.load_gather(py_v, [ej])
            zj = plsc.load_gather(pz_v, [ej])
            dx = xi - xj
            dy = yi - yj
            dz = zi - zj
            d2 = dx * dx + dy * dy + dz * dz
            ai = plsc.load_gather(atom_v, [ei])
            aj = plsc.load_gather(atom_v, [ej])
            t = ai * n_types + aj
            mul = plsc.load_gather(mulw_v, [t])
            bias = plsc.load_gather(biasw_v, [t])
            d2_v[pl.ds(off, 16)] = d2
            mul_v[pl.ds(off, 16)] = mul
            bias_v[pl.ds(off, 16)] = bias
            return carry

        lax.fori_loop(0, n_groups, body, 0)
        wb = [
            pltpu.async_copy(d2_v, d2_h.at[pl.ds(base, C)], sem),
            pltpu.async_copy(mul_v, mul_h.at[pl.ds(base, C)], sem),
            pltpu.async_copy(bias_v, bias_h.at[pl.ds(base, C)], sem),
        ]
        for cp in wb:
            cp.wait()

    return sc_gather


def _tc_rbf_chunk(d2, mul, bias, means, stds, prev, chunk_idx, n_chunks,
                  E, block_e):
    ch = d2.shape[0]
    G = means.shape[1]
    inv_a = 1.0 / math.sqrt(2.0 * math.pi)
    log2e = math.log2(math.e)
    rows = block_e // G
    nblk = ch // block_e
    nblk_total = E // block_e
    blk0 = chunk_idx * nblk

    def body(d2_ref, mul_ref, bias_ref, means_ref, stds_ref, *rest):
        out_ref, len_ref = rest[-2], rest[-1]
        length_t = jnp.sqrt(d2_ref[0])
        x_t = mul_ref[0] * length_t + bias_ref[0]
        xT = x_t.T
        std = jnp.abs(stds_ref[...]) + 1e-5
        inv = 1.0 / std
        lc = jnp.log2(inv * inv_a)
        neg_half_log2e = -0.5 * log2e
        for r in range(rows):
            col = jax.lax.slice(xT, (0, r), (G, r + 1))
            z = (col - means_ref[...]) * inv
            out_ref[pl.ds(r * G, G), :] = jnp.exp2((z * z) * neg_half_log2e + lc)
        len_ref[0] = length_t

    in_specs = [
        pl.BlockSpec((1, rows, G), lambda i: (i, 0, 0)),
        pl.BlockSpec((1, rows, G), lambda i: (i, 0, 0)),
        pl.BlockSpec((1, rows, G), lambda i: (i, 0, 0)),
        pl.BlockSpec((1, G), lambda i: (0, 0)),
        pl.BlockSpec((1, G), lambda i: (0, 0)),
    ]
    args = [d2.reshape(nblk, rows, G), mul.reshape(nblk, rows, G),
            bias.reshape(nblk, rows, G), means, stds]
    aliases = {}
    if prev is not None:
        in_specs += [pl.BlockSpec(memory_space=pl.ANY),
                     pl.BlockSpec(memory_space=pl.ANY)]
        args += [prev[0], prev[1]]
        aliases = {5: 0, 6: 1}

    return pl.pallas_call(
        body,
        grid=(nblk,),
        in_specs=in_specs,
        out_specs=[
            pl.BlockSpec((block_e, G), lambda i: (i + blk0, 0)),
            pl.BlockSpec((1, rows, G), lambda i: (i + blk0, 0, 0)),
        ],
        out_shape=[
            jax.ShapeDtypeStruct((E, G), jnp.float32),
            jax.ShapeDtypeStruct((nblk_total, rows, G), jnp.float32),
        ],
        input_output_aliases=aliases,
    )(*args)


def kernel(pos, edge_index, atom_ind, means, stds, mul_w, bias_w):
    E = edge_index.shape[1]
    n_nodes = pos.shape[0]
    n_edge_types = mul_w.shape[0]
    n_types = int(round(math.sqrt(n_edge_types)))
    n_chunks = 1
    ch = E // n_chunks
    block_e = 32000

    px, py, pz = pos[:, 0], pos[:, 1], pos[:, 2]
    mw, bw = mul_w.reshape(-1), bias_w.reshape(-1)
    sc = _make_sc_gather(ch, n_nodes, n_edge_types, n_types)

    prev = None
    for c in range(n_chunks):
        lo = c * ch
        d2, mul, bias = sc(px, py, pz, atom_ind, mw, bw,
                           jax.lax.slice(edge_index[0], (lo,), (lo + ch,)),
                           jax.lax.slice(edge_index[1], (lo,), (lo + ch,)))
        prev = _tc_rbf_chunk(d2, mul, bias, means, stds, prev, c, n_chunks,
                             E, block_e)
    out, length = prev
    return out.astype(means.dtype), length.reshape(E, 1)
